# baseline (speedup 1.0000x reference)
import jax
import jax.numpy as jnp
from jax import lax
from jax.experimental import pallas as pl
from jax.experimental.pallas import tpu as pltpu

_TEMP = 10.0


def _loss_body(lab_ref, fs_ref, ft_ref, w_ref, b_ref, out_ref, acc_s, acc_t):
    p = pl.program_id(0)
    i = pl.program_id(1)
    nb = pl.num_programs(1)
    c = w_ref.shape[1]
    rows = fs_ref.shape[0]

    @pl.when((p == 0) & (i == 0))
    def _init():
        acc_s[...] = jnp.zeros_like(acc_s)
        acc_t[...] = jnp.zeros_like(acc_t)

    w = w_ref[...]
    bias = b_ref[...]
    dn = (((1,), (0,)), ((), ()))
    ones_col = jnp.ones((rows, 1), jnp.float32)

    @pl.when(p == 0)
    def _source():
        y_s = lax.dot_general(fs_ref[...], w, dn,
                              preferred_element_type=jnp.float32) + bias
        m_s = jnp.max(y_s, axis=1, keepdims=True)
        e_s = jnp.exp((y_s - m_s) / _TEMP)
        an = e_s / jnp.sqrt(jnp.sum(e_s * e_s, axis=1, keepdims=True))
        an1 = jnp.concatenate([an, ones_col], axis=1)

        lbl = lab_ref[0]
        sub = lax.broadcasted_iota(jnp.int32, (c + 1, rows), 0)
        oh_s = (sub == lbl).astype(jnp.float32)
        acc_s[...] += lax.dot_general(
            oh_s, an1, (((1,), (0,)), ((), ())),
            preferred_element_type=jnp.float32)

    @pl.when(p == 1)
    def _target():
        y_t = lax.dot_general(ft_ref[...], w, dn,
                              preferred_element_type=jnp.float32) + bias
        m_t = jnp.max(y_t, axis=1, keepdims=True)
        s_t1 = jnp.sum(jnp.exp(y_t - m_t), axis=1, keepdims=True)
        conf = (1.0 / s_t1) > 0.8

        e_t = jnp.exp((y_t - m_t) / _TEMP)
        bn = e_t / jnp.sqrt(jnp.sum(e_t * e_t, axis=1, keepdims=True))
        bn1 = jnp.concatenate([bn, ones_col], axis=1)

        lane = lax.broadcasted_iota(jnp.int32, y_t.shape, 1)
        is_max = y_t == m_t
        first = jnp.min(jnp.where(is_max, lane, c), axis=1, keepdims=True)
        lane1 = lax.broadcasted_iota(jnp.int32, (rows, c + 1), 1)
        oh_t = ((lane1 == first) & conf).astype(jnp.float32)
        acc_t[...] += lax.dot_general(
            oh_t, bn1, (((0,), (0,)), ((), ())),
            preferred_element_type=jnp.float32)

    @pl.when((p == 1) & (i == nb - 1))
    def _finish():
        prod = acc_s[...] * acc_t[...]
        total = jnp.sum(prod)
        lane2 = lax.broadcasted_iota(jnp.int32, prod.shape, 1)
        count = jnp.sum(jnp.where(lane2 == c, prod, 0.0))
        loss_sum = 2.0 * count - total
        out_ref[0, 0] = (_TEMP * _TEMP) * loss_sum / jnp.maximum(count, 1.0)


def kernel(f_s, f_t, label_s, W, b):
    n, d = f_s.shape
    c = W.shape[1]
    rows = 512
    nb = n // rows

    lab3 = label_s.reshape(nb, 1, rows)
    b2 = b.reshape(1, c)

    out = pl.pallas_call(
        _loss_body,
        grid=(2, nb),
        in_specs=[
            pl.BlockSpec((1, 1, rows), lambda p, i: (i, 0, 0)),
            pl.BlockSpec((rows, d), lambda p, i, nb=nb: (i * (1 - p) + (nb - 1) * p, 0)),
            pl.BlockSpec((rows, d), lambda p, i: (i * p, 0)),
            pl.BlockSpec((d, c), lambda p, i: (0, 0)),
            pl.BlockSpec((1, c), lambda p, i: (0, 0)),
        ],
        out_specs=pl.BlockSpec(memory_space=pltpu.SMEM),
        out_shape=jax.ShapeDtypeStruct((1, 1), jnp.float32),
        scratch_shapes=[
            pltpu.VMEM((c + 1, c + 1), jnp.float32),
            pltpu.VMEM((c + 1, c + 1), jnp.float32),
        ],
    )(lab3, f_s, f_t, W, b2)
    return out.reshape(1)


# speedup vs baseline: 1.0727x; 1.0727x over previous
import jax
import jax.numpy as jnp
from jax import lax
from jax.experimental import pallas as pl
from jax.experimental.pallas import tpu as pltpu

_TEMP = 10.0


def _loss_body(lab_ref, fs_ref, ft_ref, w_ref, b_ref, out_ref, acc_s, acc_t):
    i = pl.program_id(0)
    nb = pl.num_programs(0) - 1
    c = w_ref.shape[1]
    rows = fs_ref.shape[0]

    @pl.when(i == 0)
    def _init():
        acc_s[...] = jnp.zeros_like(acc_s)
        acc_t[...] = jnp.zeros_like(acc_t)

    w = w_ref[...]
    bias = b_ref[...]
    dn = (((1,), (0,)), ((), ()))
    ones_col = jnp.ones((rows, 1), jnp.float32)

    @pl.when(i < nb)
    def _source():
        y_s = lax.dot_general(fs_ref[...], w, dn,
                              preferred_element_type=jnp.float32) + bias
        m_s = jnp.max(y_s, axis=1, keepdims=True)
        e_s = jnp.exp((y_s - m_s) / _TEMP)
        an = e_s / jnp.sqrt(jnp.sum(e_s * e_s, axis=1, keepdims=True))
        an1 = jnp.concatenate([an, ones_col], axis=1)

        lbl = lab_ref[0]
        sub = lax.broadcasted_iota(jnp.int32, (c + 1, rows), 0)
        oh_s = (sub == lbl).astype(jnp.float32)
        acc_s[...] += lax.dot_general(
            oh_s, an1, (((1,), (0,)), ((), ())),
            preferred_element_type=jnp.float32)

    @pl.when(i > 0)
    def _target():
        y_t = lax.dot_general(ft_ref[...], w, dn,
                              preferred_element_type=jnp.float32) + bias
        m_t = jnp.max(y_t, axis=1, keepdims=True)
        s_t1 = jnp.sum(jnp.exp(y_t - m_t), axis=1, keepdims=True)
        conf = (1.0 / s_t1) > 0.8

        e_t = jnp.exp((y_t - m_t) / _TEMP)
        bn = e_t / jnp.sqrt(jnp.sum(e_t * e_t, axis=1, keepdims=True))
        bn1 = jnp.concatenate([bn, ones_col], axis=1)

        lane = lax.broadcasted_iota(jnp.int32, y_t.shape, 1)
        is_max = y_t == m_t
        first = jnp.min(jnp.where(is_max, lane, c), axis=1, keepdims=True)
        lane1 = lax.broadcasted_iota(jnp.int32, (rows, c + 1), 1)
        oh_t = ((lane1 == first) & conf).astype(jnp.float32)
        acc_t[...] += lax.dot_general(
            oh_t, bn1, (((0,), (0,)), ((), ())),
            preferred_element_type=jnp.float32)

    @pl.when(i == nb)
    def _finish():
        prod = acc_s[...] * acc_t[...]
        total = jnp.sum(prod)
        lane2 = lax.broadcasted_iota(jnp.int32, prod.shape, 1)
        count = jnp.sum(jnp.where(lane2 == c, prod, 0.0))
        loss_sum = 2.0 * count - total
        out_ref[0, 0] = (_TEMP * _TEMP) * loss_sum / jnp.maximum(count, 1.0)


def kernel(f_s, f_t, label_s, W, b):
    n, d = f_s.shape
    c = W.shape[1]
    rows = 512
    nb = n // rows

    lab3 = label_s.reshape(nb, 1, rows)
    b2 = b.reshape(1, c)

    out = pl.pallas_call(
        _loss_body,
        grid=(nb + 1,),
        in_specs=[
            pl.BlockSpec((1, 1, rows), lambda i, nb=nb: (jnp.minimum(i, nb - 1), 0, 0)),
            pl.BlockSpec((rows, d), lambda i, nb=nb: (jnp.minimum(i, nb - 1), 0)),
            pl.BlockSpec((rows, d), lambda i: (jnp.maximum(i - 1, 0), 0)),
            pl.BlockSpec((d, c), lambda i: (0, 0)),
            pl.BlockSpec((1, c), lambda i: (0, 0)),
        ],
        out_specs=pl.BlockSpec(memory_space=pltpu.SMEM),
        out_shape=jax.ShapeDtypeStruct((1, 1), jnp.float32),
        scratch_shapes=[
            pltpu.VMEM((c + 1, c + 1), jnp.float32),
            pltpu.VMEM((c + 1, c + 1), jnp.float32),
        ],
    )(lab3, f_s, f_t, W, b2)
    return out.reshape(1)
